# baseline (speedup 1.0000x reference)
import functools

import jax
import jax.numpy as jnp
from jax import lax
from jax.experimental import pallas as pl
from jax.experimental.pallas import tpu as pltpu
from jax.experimental.pallas import tpu_sc as plsc

F = 26
B = 4096
V = 100000
D = 64

NC = 2
NS = 16
NW = NC * NS
ROWS_PER_W = F * B // NW
CHUNK = 128
NCHUNK = ROWS_PER_W // CHUNK

_mesh = plsc.VectorSubcoreMesh(core_axis_name="c", subcore_axis_name="s")


@functools.partial(
    pl.kernel,
    mesh=_mesh,
    out_type=jax.ShapeDtypeStruct((F * B, D), jnp.float32),
    scratch_types=[
        pltpu.VMEM((NCHUNK, CHUNK), jnp.int32),
        pltpu.VMEM((CHUNK, D), jnp.float32),
        pltpu.SemaphoreType.DMA,
    ],
    compiler_params=pltpu.CompilerParams(use_tc_tiling_on_sc=False),
)
def _gather_kernel(values_hbm, tables_hbm, out_hbm, idx_v, rows_v, sem):
    wid = lax.axis_index("s") * NC + lax.axis_index("c")
    base = wid * ROWS_PER_W
    pltpu.sync_copy(values_hbm.at[wid], idx_v)

    def body(j, carry):
        off = ((base + j * CHUNK) // B) * V
        for s in range(CHUNK // 16):
            sl = pl.ds(s * 16, 16)
            idx_v[j, sl] = idx_v[j, sl] + off
        pltpu.async_copy(tables_hbm.at[idx_v.at[j]], rows_v, sem).wait()
        pltpu.sync_copy(rows_v, out_hbm.at[pl.ds(base + j * CHUNK, CHUNK)])
        return carry

    lax.fori_loop(0, NCHUNK, body, 0)


def kernel(values, lengths, tables):
    del lengths
    vals = values.reshape(NW, NCHUNK, CHUNK)
    tabs = tables.reshape(F * V, D)
    out = _gather_kernel(vals, tabs)
    return out.reshape(F, B, D)


# speedup vs baseline: 1.0108x; 1.0108x over previous
import functools

import jax
import jax.numpy as jnp
from jax import lax
from jax.experimental import pallas as pl
from jax.experimental.pallas import tpu as pltpu
from jax.experimental.pallas import tpu_sc as plsc

F = 26
B = 4096
V = 100000
D = 64

NC = 2
NS = 16
NW = NC * NS
ROWS_PER_W = F * B // NW
CHUNK = 128
NCHUNK = ROWS_PER_W // CHUNK
M = 8

_mesh = plsc.VectorSubcoreMesh(core_axis_name="c", subcore_axis_name="s")


@functools.partial(
    pl.kernel,
    mesh=_mesh,
    out_type=jax.ShapeDtypeStruct((F * B, D), jnp.float32),
    scratch_types=[
        pltpu.VMEM((NCHUNK, CHUNK), jnp.int32),
        pltpu.VMEM((M, CHUNK, D), jnp.float32),
        [pltpu.SemaphoreType.DMA] * M,
        [pltpu.SemaphoreType.DMA] * M,
    ],
    compiler_params=pltpu.CompilerParams(use_tc_tiling_on_sc=False),
)
def _gather_kernel(values_hbm, tables_hbm, out_hbm, idx_v, rows_v, gsem, ssem):
    wid = lax.axis_index("s") * NC + lax.axis_index("c")
    base = wid * ROWS_PER_W
    pltpu.sync_copy(values_hbm.at[wid], idx_v)

    def fix(j):
        off = ((base + j * CHUNK) // B) * V
        for s in range(CHUNK // 16):
            sl = pl.ds(s * 16, 16)
            idx_v[j, sl] = idx_v[j, sl] + off

    def fire_gather(j):
        return pltpu.async_copy(
            tables_hbm.at[idx_v.at[j]], rows_v.at[j % M], gsem[j % M]
        )

    def fire_store(j):
        return pltpu.async_copy(
            rows_v.at[j % M], out_hbm.at[pl.ds(base + j * CHUNK, CHUNK)],
            ssem[j % M],
        )

    gh = {}
    sh = {}
    for j in range(M):
        fix(j)
        gh[j] = fire_gather(j)
    for j in range(NCHUNK):
        gh[j].wait()
        sh[j] = fire_store(j)
        nj = j + M
        if nj < NCHUNK:
            sh[j].wait()
            fix(nj)
            gh[nj] = fire_gather(nj)
    for j in range(NCHUNK - M, NCHUNK):
        sh[j].wait()


def kernel(values, lengths, tables):
    del lengths
    vals = values.reshape(NW, NCHUNK, CHUNK)
    tabs = tables.reshape(F * V, D)
    out = _gather_kernel(vals, tabs)
    return out.reshape(F, B, D)


# speedup vs baseline: 3.9654x; 3.9232x over previous
import functools

import jax
import jax.numpy as jnp
from jax import lax
from jax.experimental import pallas as pl
from jax.experimental.pallas import tpu as pltpu
from jax.experimental.pallas import tpu_sc as plsc

F = 26
B = 4096
V = 100000
D = 64

NW = 32
WIN = 4096
NWIN = 25
VMAIN = 99968
TAIL = VMAIN - 24 * WIN
VT = V - VMAIN
LANES = 16
BUFN = B + NWIN * (LANES - 1) + 9

_mesh = plsc.VectorSubcoreMesh(core_axis_name="c", subcore_axis_name="s")


def _splat(x):
    return jnp.full((LANES,), x, jnp.int32)


def _scalar_at(ref, i):
    return lax.reduce_max(plsc.load_gather(ref, [_splat(i)]), (0,))


@functools.partial(
    pl.kernel,
    mesh=_mesh,
    out_type=jax.ShapeDtypeStruct((F, D, B), jnp.float32),
    scratch_types=[
        pltpu.VMEM((B,), jnp.int32),
        pltpu.VMEM((BUFN,), jnp.int32),
        pltpu.VMEM((BUFN,), jnp.int32),
        pltpu.VMEM((32,), jnp.int32),
        pltpu.VMEM((32,), jnp.int32),
        pltpu.VMEM((32,), jnp.int32),
        pltpu.VMEM((8, WIN), jnp.float32),
        pltpu.VMEM((8, WIN), jnp.float32),
        pltpu.VMEM((VT * D,), jnp.float32),
        pltpu.VMEM((8, B), jnp.float32),
        pltpu.SemaphoreType.DMA,
        pltpu.SemaphoreType.DMA,
    ],
    compiler_params=pltpu.CompilerParams(
        use_tc_tiling_on_sc=True, needs_layout_passes=False
    ),
)
def _gather_kernel(values_hbm, tabs_hbm, tail_hbm, out_hbm, idxf, vbuf, bbuf,
                   cnt_v, start_v, fill_v, chunk0, chunk1, tail_v, outb,
                   sem0, sem1):
    wid = lax.axis_index("s") * 2 + lax.axis_index("c")
    s_lo = (13 * wid) // 2
    s_hi = (13 * (wid + 1)) // 2
    iota = lax.iota(jnp.int32, LANES)
    zeros = jnp.zeros((LANES,), jnp.int32)

    chunks = (chunk0, chunk1)
    sems = (sem0, sem1)

    def slab_body(s, carry):
        f = s >> 3
        d0 = (s & 7) * 8

        pltpu.sync_copy(values_hbm.at[pl.ds(f * B, B)], idxf)
        pltpu.sync_copy(tail_hbm.at[pl.ds(f * (VT * D), VT * D)], tail_v)

        h0 = pltpu.async_copy(
            tabs_hbm.at[f, pl.ds(d0, 8), pl.ds(0, WIN)], chunk0, sem0)
        h1 = pltpu.async_copy(
            tabs_hbm.at[f, pl.ds(d0, 8), pl.ds(WIN, WIN)], chunk1, sem1)
        handles = {0: h0, 1: h1}

        cnt_v[pl.ds(0, 16)] = zeros
        cnt_v[pl.ds(16, 16)] = zeros

        def hist(i, c):
            v = idxf[pl.ds(i * LANES, LANES)]
            w = (v >> 12) + 1
            rank, last = plsc.scan_count(w)
            plsc.addupdate_scatter(cnt_v, [w], rank, mask=last)
            return c

        lax.fori_loop(0, B // LANES, hist, 0)

        c0 = cnt_v[pl.ds(0, 16)]
        c1 = cnt_v[pl.ds(16, 16)]
        r0 = (c0 + (LANES - 1)) & ~(LANES - 1)
        r1 = (c1 + (LANES - 1)) & ~(LANES - 1)
        s0 = plsc.cumsum(r0) - r0
        s1 = plsc.cumsum(r1) - r1 + lax.reduce_sum(r0, (0,))
        start_v[pl.ds(0, 16)] = s0
        start_v[pl.ds(16, 16)] = s1
        fill_v[pl.ds(0, 16)] = c0 * 0 + s0
        fill_v[pl.ds(16, 16)] = c1 * 0 + s1

        def scat(i, c):
            v = idxf[pl.ds(i * LANES, LANES)]
            b = i * LANES + iota
            w = (v >> 12) + 1
            rank, last = plsc.scan_count(w)
            base = plsc.load_gather(fill_v, [w])
            pos = base + rank - 1
            plsc.store_scatter(vbuf, [pos], v)
            plsc.store_scatter(bbuf, [pos], b)
            plsc.addupdate_scatter(fill_v, [w], rank, mask=last)
            return c

        lax.fori_loop(0, B // LANES, scat, 0)

        for w in range(NWIN):
            buf = chunks[w % 2]
            handles[w % 2].wait()
            start = _scalar_at(start_v, w + 1)
            n = _scalar_at(cnt_v, w + 1)
            start = jnp.minimum(start, BUFN - LANES)
            n = jnp.minimum(n, B)

            def pull(j, c, buf=buf, w=w, start=start, n=n):
                k = jnp.minimum(start + j * LANES, BUFN - LANES)
                v = vbuf[pl.ds(k, LANES)]
                b = bbuf[pl.ds(k, LANES)]
                m = (j * LANES + iota) < n
                b = b & (B - 1)
                vrel = v - w * WIN
                b = jnp.where(m, b, 0)
                if w < NWIN - 1:
                    vrel = jnp.where(m, vrel, 0)
                    for dl in range(8):
                        val = plsc.load_gather(buf, [_splat(dl), vrel], mask=m)
                        plsc.store_scatter(outb, [_splat(dl), b], val, mask=m)
                else:
                    m_in = m & (vrel < TAIL)
                    m_t = m & (vrel >= TAIL)
                    vin = jnp.where(m_in, vrel, 0)
                    vt = jnp.where(m_t, (v - VMAIN) * D + d0, 0)
                    for dl in range(8):
                        val = plsc.load_gather(
                            buf, [_splat(dl), vin], mask=m_in)
                        plsc.store_scatter(outb, [_splat(dl), b], val,
                                           mask=m_in)
                        tval = plsc.load_gather(tail_v, [vt + dl], mask=m_t)
                        plsc.store_scatter(outb, [_splat(dl), b], tval,
                                           mask=m_t)
                return c

            lax.fori_loop(0, (n + LANES - 1) >> 4, pull, 0)

            nxt = w + 2
            if nxt < NWIN:
                ext = WIN if nxt < NWIN - 1 else TAIL
                handles[nxt % 2] = pltpu.async_copy(
                    tabs_hbm.at[f, pl.ds(d0, 8), pl.ds(nxt * WIN, ext)],
                    chunks[nxt % 2].at[:, pl.ds(0, ext)],
                    sems[nxt % 2],
                )

        pltpu.sync_copy(outb, out_hbm.at[f, pl.ds(d0, 8), pl.ds(0, B)])
        return carry

    lax.fori_loop(s_lo, s_hi, slab_body, 0)


def kernel(values, lengths, tables):
    del lengths
    tabs_t = tables.transpose(0, 2, 1)
    tail = tables[:, VMAIN:, :].reshape(F * VT * D)
    vals = values.reshape(F * B)
    out = _gather_kernel(vals, tabs_t, tail)
    return out.transpose(0, 2, 1)


# speedup vs baseline: 4.3039x; 1.0854x over previous
import functools

import jax
import jax.numpy as jnp
from jax import lax
from jax.experimental import pallas as pl
from jax.experimental.pallas import tpu as pltpu
from jax.experimental.pallas import tpu_sc as plsc

F = 26
B = 4096
V = 100000
D = 64

NW = 32
WIN = 2048
NWIN = 49
VMAIN = 99968
TAIL = VMAIN - (NWIN - 1) * WIN
VT = V - VMAIN
LANES = 16
BUFN = B + NWIN * (LANES - 1) + 1

_mesh = plsc.VectorSubcoreMesh(core_axis_name="c", subcore_axis_name="s")


def _splat(x):
    return jnp.full((LANES,), x, jnp.int32)


def _scalar_at(ref, i):
    return lax.reduce_max(plsc.load_gather(ref, [_splat(i)]), (0,))


@functools.partial(
    pl.kernel,
    mesh=_mesh,
    out_type=jax.ShapeDtypeStruct((F, D, B), jnp.float32),
    scratch_types=[
        pltpu.VMEM((B,), jnp.int32),
        pltpu.VMEM((BUFN,), jnp.int32),
        pltpu.VMEM((BUFN,), jnp.int32),
        pltpu.VMEM((64,), jnp.int32),
        pltpu.VMEM((64,), jnp.int32),
        pltpu.VMEM((64,), jnp.int32),
        pltpu.VMEM((8, WIN), jnp.float32),
        pltpu.VMEM((8, WIN), jnp.float32),
        pltpu.VMEM((8, WIN), jnp.float32),
        pltpu.VMEM((VT * D,), jnp.float32),
        pltpu.VMEM((8, B), jnp.float32),
        pltpu.SemaphoreType.DMA,
        pltpu.SemaphoreType.DMA,
        pltpu.SemaphoreType.DMA,
    ],
    compiler_params=pltpu.CompilerParams(
        use_tc_tiling_on_sc=True, needs_layout_passes=False
    ),
)
def _gather_kernel(values_hbm, tabs_hbm, tail_hbm, out_hbm, idxf, vbuf, bbuf,
                   cnt_v, start_v, fill_v, chunk0, chunk1, chunk2, tail_v,
                   outb, sem0, sem1, sem2):
    wid = lax.axis_index("s") * 2 + lax.axis_index("c")
    s_lo = (13 * wid) // 2
    s_hi = (13 * (wid + 1)) // 2
    f_lo = s_lo >> 3
    f_hi = (s_hi + 7) >> 3
    iota = lax.iota(jnp.int32, LANES)
    zeros = jnp.zeros((LANES,), jnp.int32)

    chunks = (chunk0, chunk1, chunk2)
    sems = (sem0, sem1, sem2)

    def feature_body(f, carry):
        pltpu.sync_copy(values_hbm.at[pl.ds(f * B, B)], idxf)
        pltpu.sync_copy(tail_hbm.at[pl.ds(f * (VT * D), VT * D)], tail_v)

        for q in range(4):
            cnt_v[pl.ds(q * 16, 16)] = zeros

        def hist(i, c):
            for u in range(4):
                v = idxf[pl.ds((i * 4 + u) * LANES, LANES)]
                w = (v >> 11) + 1
                rank, last = plsc.scan_count(w)
                plsc.addupdate_scatter(cnt_v, [w], rank, mask=last)
            return c

        lax.fori_loop(0, B // LANES // 4, hist, 0)

        tot = 0
        for q in range(4):
            cq = cnt_v[pl.ds(q * 16, 16)]
            rq = (cq + (LANES - 1)) & ~(LANES - 1)
            sq = plsc.cumsum(rq) - rq + tot
            start_v[pl.ds(q * 16, 16)] = sq
            fill_v[pl.ds(q * 16, 16)] = cq * 0 + sq
            tot = tot + lax.reduce_sum(rq, (0,))

        def scat(i, c):
            for u in range(4):
                j = i * 4 + u
                v = idxf[pl.ds(j * LANES, LANES)]
                b = j * LANES + iota
                w = (v >> 11) + 1
                rank, last = plsc.scan_count(w)
                base = plsc.load_gather(fill_v, [w])
                pos = base + rank - 1
                plsc.store_scatter(vbuf, [pos], v)
                plsc.store_scatter(bbuf, [pos], b)
                plsc.addupdate_scatter(fill_v, [w], rank, mask=last)
            return c

        lax.fori_loop(0, B // LANES // 4, scat, 0)

        dhi_lo = jnp.maximum(s_lo - f * 8, 0)
        dhi_hi = jnp.minimum(s_hi - f * 8, 8)

        def slab_body(dhi, carry2):
            d0 = dhi * 8
            h0 = pltpu.async_copy(
                tabs_hbm.at[f, pl.ds(d0, 8), pl.ds(0, WIN)], chunk0, sem0)
            h1 = pltpu.async_copy(
                tabs_hbm.at[f, pl.ds(d0, 8), pl.ds(WIN, WIN)], chunk1, sem1)
            handles = {0: h0, 1: h1}

            for w in range(NWIN):
                buf = chunks[w % 3]
                handles[w % 3].wait()
                nxt = w + 2
                if nxt < NWIN:
                    ext = WIN if nxt < NWIN - 1 else TAIL
                    handles[nxt % 3] = pltpu.async_copy(
                        tabs_hbm.at[f, pl.ds(d0, 8), pl.ds(nxt * WIN, ext)],
                        chunks[nxt % 3].at[:, pl.ds(0, ext)],
                        sems[nxt % 3],
                    )
                start = _scalar_at(start_v, w + 1)
                n = _scalar_at(cnt_v, w + 1)
                start = jnp.minimum(start, BUFN - LANES)
                n = jnp.minimum(n, B)

                def pull(j, c, buf=buf, w=w, start=start, n=n):
                    k = jnp.minimum(start + j * LANES, BUFN - LANES)
                    v = vbuf[pl.ds(k, LANES)]
                    b = bbuf[pl.ds(k, LANES)]
                    m = (j * LANES + iota) < n
                    b = jnp.where(m, b & (B - 1), 0)
                    vrel = v - w * WIN
                    if w < NWIN - 1:
                        vrel = jnp.where(m, vrel, 0)
                        for dl in range(8):
                            val = plsc.load_gather(
                                buf, [_splat(dl), vrel], mask=m)
                            plsc.store_scatter(
                                outb, [_splat(dl), b], val, mask=m)
                    else:
                        m_in = m & (vrel < TAIL)
                        m_t = m & (vrel >= TAIL)
                        vin = jnp.where(m_in, vrel, 0)
                        vt = jnp.where(m_t, (v - VMAIN) * D + d0, 0)
                        for dl in range(8):
                            val = plsc.load_gather(
                                buf, [_splat(dl), vin], mask=m_in)
                            plsc.store_scatter(
                                outb, [_splat(dl), b], val, mask=m_in)
                            tval = plsc.load_gather(
                                tail_v, [vt + dl], mask=m_t)
                            plsc.store_scatter(
                                outb, [_splat(dl), b], tval, mask=m_t)
                    return c

                lax.fori_loop(0, (n + LANES - 1) >> 4, pull, 0)

            pltpu.sync_copy(outb, out_hbm.at[f, pl.ds(d0, 8), pl.ds(0, B)])
            return carry2

        lax.fori_loop(dhi_lo, dhi_hi, slab_body, 0)
        return carry

    lax.fori_loop(f_lo, f_hi, feature_body, 0)


def kernel(values, lengths, tables):
    del lengths
    tabs_t = tables.transpose(0, 2, 1)
    tail = tables[:, VMAIN:, :].reshape(F * VT * D)
    vals = values.reshape(F * B)
    out = _gather_kernel(vals, tabs_t, tail)
    return out.transpose(0, 2, 1)


# speedup vs baseline: 4.5569x; 1.0588x over previous
import functools

import jax
import jax.numpy as jnp
from jax import lax
from jax.experimental import pallas as pl
from jax.experimental.pallas import tpu as pltpu
from jax.experimental.pallas import tpu_sc as plsc

F = 26
B = 4096
V = 100000
D = 64

NW = 32
WIN = 3072
NWIN = 33
VMAIN = 99968
TAIL = VMAIN - (NWIN - 1) * WIN
VT = V - VMAIN
LANES = 16
BUFN = B + NWIN * (LANES - 1) + 1

_mesh = plsc.VectorSubcoreMesh(core_axis_name="c", subcore_axis_name="s")


def _splat(x):
    return jnp.full((LANES,), x, jnp.int32)


def _scalar_at(ref, i):
    return lax.reduce_max(plsc.load_gather(ref, [_splat(i)]), (0,))


@functools.partial(
    pl.kernel,
    mesh=_mesh,
    out_type=jax.ShapeDtypeStruct((F, D, B), jnp.float32),
    scratch_types=[
        pltpu.VMEM((B,), jnp.int32),
        pltpu.VMEM((BUFN,), jnp.int32),
        pltpu.VMEM((BUFN,), jnp.int32),
        pltpu.VMEM((64,), jnp.int32),
        pltpu.VMEM((64,), jnp.int32),
        pltpu.VMEM((64,), jnp.int32),
        pltpu.VMEM((8, WIN), jnp.float32),
        pltpu.VMEM((8, WIN), jnp.float32),
        pltpu.VMEM((8, WIN), jnp.float32),
        pltpu.VMEM((VT * D,), jnp.float32),
        pltpu.VMEM((8, B), jnp.float32),
        pltpu.SemaphoreType.DMA,
        pltpu.SemaphoreType.DMA,
        pltpu.SemaphoreType.DMA,
    ],
    compiler_params=pltpu.CompilerParams(
        use_tc_tiling_on_sc=True, needs_layout_passes=False
    ),
)
def _gather_kernel(values_hbm, tabs_hbm, tail_hbm, out_hbm, idxf, vbuf, bbuf,
                   cnt_v, start_v, fill_v, chunk0, chunk1, chunk2, tail_v,
                   outb, sem0, sem1, sem2):
    wid = lax.axis_index("s") * 2 + lax.axis_index("c")
    s_lo = (13 * wid) // 2
    s_hi = (13 * (wid + 1)) // 2
    f_lo = s_lo >> 3
    f_hi = (s_hi + 7) >> 3
    iota = lax.iota(jnp.int32, LANES)
    zeros = jnp.zeros((LANES,), jnp.int32)

    chunks = (chunk0, chunk1, chunk2)
    sems = (sem0, sem1, sem2)

    def feature_body(f, carry):
        pltpu.sync_copy(values_hbm.at[pl.ds(f * B, B)], idxf)
        pltpu.sync_copy(tail_hbm.at[pl.ds(f * (VT * D), VT * D)], tail_v)

        for q in range(4):
            cnt_v[pl.ds(q * 16, 16)] = zeros

        def hist(i, c):
            for u in range(4):
                v = idxf[pl.ds((i * 4 + u) * LANES, LANES)]
                w = (((v >> 10) * 21846) >> 16) + 1
                rank, last = plsc.scan_count(w)
                plsc.addupdate_scatter(cnt_v, [w], rank, mask=last)
            return c

        lax.fori_loop(0, B // LANES // 4, hist, 0)

        tot = 0
        for q in range(4):
            cq = cnt_v[pl.ds(q * 16, 16)]
            rq = (cq + (LANES - 1)) & ~(LANES - 1)
            sq = plsc.cumsum(rq) - rq + tot
            start_v[pl.ds(q * 16, 16)] = sq
            fill_v[pl.ds(q * 16, 16)] = cq * 0 + sq
            tot = tot + lax.reduce_sum(rq, (0,))

        def scat(i, c):
            for u in range(4):
                j = i * 4 + u
                v = idxf[pl.ds(j * LANES, LANES)]
                b = j * LANES + iota
                w = (((v >> 10) * 21846) >> 16) + 1
                rank, last = plsc.scan_count(w)
                base = plsc.load_gather(fill_v, [w])
                pos = base + rank - 1
                plsc.store_scatter(vbuf, [pos], v)
                plsc.store_scatter(bbuf, [pos], b)
                plsc.addupdate_scatter(fill_v, [w], rank, mask=last)
            return c

        lax.fori_loop(0, B // LANES // 4, scat, 0)

        dhi_lo = jnp.maximum(s_lo - f * 8, 0)
        dhi_hi = jnp.minimum(s_hi - f * 8, 8)

        def slab_body(dhi, carry2):
            d0 = dhi * 8
            h0 = pltpu.async_copy(
                tabs_hbm.at[f, pl.ds(d0, 8), pl.ds(0, WIN)], chunk0, sem0)
            h1 = pltpu.async_copy(
                tabs_hbm.at[f, pl.ds(d0, 8), pl.ds(WIN, WIN)], chunk1, sem1)
            handles = {0: h0, 1: h1}

            for w in range(NWIN):
                buf = chunks[w % 3]
                handles[w % 3].wait()
                nxt = w + 2
                if nxt < NWIN:
                    ext = WIN if nxt < NWIN - 1 else TAIL
                    handles[nxt % 3] = pltpu.async_copy(
                        tabs_hbm.at[f, pl.ds(d0, 8), pl.ds(nxt * WIN, ext)],
                        chunks[nxt % 3].at[:, pl.ds(0, ext)],
                        sems[nxt % 3],
                    )
                start = _scalar_at(start_v, w + 1)
                n = _scalar_at(cnt_v, w + 1)
                start = jnp.minimum(start, BUFN - LANES)
                n = jnp.minimum(n, B)

                def pull(j, c, buf=buf, w=w, start=start, n=n):
                    k = jnp.minimum(start + j * LANES, BUFN - LANES)
                    v = vbuf[pl.ds(k, LANES)]
                    b = bbuf[pl.ds(k, LANES)]
                    m = (j * LANES + iota) < n
                    b = jnp.where(m, b & (B - 1), 0)
                    vrel = v - w * WIN
                    if w < NWIN - 1:
                        vrel = jnp.where(m, vrel, 0)
                        for dl in range(8):
                            val = plsc.load_gather(
                                buf, [_splat(dl), vrel], mask=m)
                            plsc.store_scatter(
                                outb, [_splat(dl), b], val, mask=m)
                    else:
                        m_in = m & (vrel < TAIL)
                        m_t = m & (vrel >= TAIL)
                        vin = jnp.where(m_in, vrel, 0)
                        vt = jnp.where(m_t, (v - VMAIN) * D + d0, 0)
                        for dl in range(8):
                            val = plsc.load_gather(
                                buf, [_splat(dl), vin], mask=m_in)
                            plsc.store_scatter(
                                outb, [_splat(dl), b], val, mask=m_in)
                            tval = plsc.load_gather(
                                tail_v, [vt + dl], mask=m_t)
                            plsc.store_scatter(
                                outb, [_splat(dl), b], tval, mask=m_t)
                    return c

                lax.fori_loop(0, (n + LANES - 1) >> 4, pull, 0)

            pltpu.sync_copy(outb, out_hbm.at[f, pl.ds(d0, 8), pl.ds(0, B)])
            return carry2

        lax.fori_loop(dhi_lo, dhi_hi, slab_body, 0)
        return carry

    lax.fori_loop(f_lo, f_hi, feature_body, 0)


def kernel(values, lengths, tables):
    del lengths
    tabs_t = tables.transpose(0, 2, 1)
    tail = tables[:, VMAIN:, :].reshape(F * VT * D)
    vals = values.reshape(F * B)
    out = _gather_kernel(vals, tabs_t, tail)
    return out.transpose(0, 2, 1)


# speedup vs baseline: 4.7948x; 1.0522x over previous
import functools

import jax
import jax.numpy as jnp
from jax import lax
from jax.experimental import pallas as pl
from jax.experimental.pallas import tpu as pltpu
from jax.experimental.pallas import tpu_sc as plsc

F = 26
B = 4096
V = 100000
D = 64

NW = 32
WIN = 3072
NWIN = 33
VMAIN = 99968
TAIL = VMAIN - (NWIN - 1) * WIN
VT = V - VMAIN
LANES = 16
BUFN = B + NWIN * (LANES - 1) + 1

_mesh = plsc.VectorSubcoreMesh(core_axis_name="c", subcore_axis_name="s")


def _splat(x):
    return jnp.full((LANES,), x, jnp.int32)


def _scalar_at(ref, i):
    return lax.reduce_max(plsc.load_gather(ref, [_splat(i)]), (0,))


@functools.partial(
    pl.kernel,
    mesh=_mesh,
    out_type=jax.ShapeDtypeStruct((F, D, B), jnp.float32),
    scratch_types=[
        pltpu.VMEM((B,), jnp.int32),
        pltpu.VMEM((BUFN,), jnp.int32),
        pltpu.VMEM((BUFN,), jnp.int32),
        pltpu.VMEM((64,), jnp.int32),
        pltpu.VMEM((64,), jnp.int32),
        pltpu.VMEM((64,), jnp.int32),
        pltpu.VMEM((8, WIN), jnp.float32),
        pltpu.VMEM((8, WIN), jnp.float32),
        pltpu.VMEM((8, WIN), jnp.float32),
        pltpu.VMEM((VT * D,), jnp.float32),
        pltpu.VMEM((8, B), jnp.float32),
        pltpu.SemaphoreType.DMA,
        pltpu.SemaphoreType.DMA,
        pltpu.SemaphoreType.DMA,
    ],
    compiler_params=pltpu.CompilerParams(
        use_tc_tiling_on_sc=True, needs_layout_passes=False
    ),
)
def _gather_kernel(values_hbm, tabs_hbm, tail_hbm, out_hbm, idxf, vbuf, bbuf,
                   cnt_v, start_v, fill_v, chunk0, chunk1, chunk2, tail_v,
                   outb, sem0, sem1, sem2):
    wid = lax.axis_index("s") * 2 + lax.axis_index("c")
    s_lo = (13 * wid) // 2
    s_hi = (13 * (wid + 1)) // 2
    f_lo = s_lo >> 3
    f_hi = (s_hi + 7) >> 3
    iota = lax.iota(jnp.int32, LANES)
    zeros = jnp.zeros((LANES,), jnp.int32)

    chunks = (chunk0, chunk1, chunk2)
    sems = (sem0, sem1, sem2)

    def fire(fv, d0v, w, slot):
        ext = WIN if w < NWIN - 1 else TAIL
        pltpu.async_copy(
            tabs_hbm.at[fv, pl.ds(d0v, 8), pl.ds(w * WIN, ext)],
            chunks[slot].at[:, pl.ds(0, ext)], sems[slot])

    def wait_win(w):
        ext = WIN if w < NWIN - 1 else TAIL
        slot = w % 3
        pltpu.make_async_copy(
            tabs_hbm.at[0, pl.ds(0, 8), pl.ds(0, ext)],
            chunks[slot].at[:, pl.ds(0, ext)], sems[slot]).wait()

    def feature_body(f, carry):
        dhi_lo = jnp.maximum(s_lo - f * 8, 0)
        dhi_hi = jnp.minimum(s_hi - f * 8, 8)

        fire(f, dhi_lo * 8, 0, 0)
        fire(f, dhi_lo * 8, 1, 1)

        pltpu.sync_copy(values_hbm.at[pl.ds(f * B, B)], idxf)
        pltpu.sync_copy(tail_hbm.at[pl.ds(f * (VT * D), VT * D)], tail_v)

        for q in range(4):
            cnt_v[pl.ds(q * 16, 16)] = zeros

        def hist(i, c):
            for u in range(4):
                v = idxf[pl.ds((i * 4 + u) * LANES, LANES)]
                w = (((v >> 10) * 21846) >> 16) + 1
                rank, last = plsc.scan_count(w)
                plsc.addupdate_scatter(cnt_v, [w], rank, mask=last)
            return c

        lax.fori_loop(0, B // LANES // 4, hist, 0)

        tot = 0
        for q in range(4):
            cq = cnt_v[pl.ds(q * 16, 16)]
            rq = (cq + (LANES - 1)) & ~(LANES - 1)
            sq = plsc.cumsum(rq) - rq + tot
            start_v[pl.ds(q * 16, 16)] = sq
            fill_v[pl.ds(q * 16, 16)] = cq * 0 + sq
            tot = tot + lax.reduce_sum(rq, (0,))

        def scat(i, c):
            for u in range(4):
                j = i * 4 + u
                v = idxf[pl.ds(j * LANES, LANES)]
                b = j * LANES + iota
                w = (((v >> 10) * 21846) >> 16) + 1
                rank, last = plsc.scan_count(w)
                base = plsc.load_gather(fill_v, [w])
                pos = base + rank - 1
                plsc.store_scatter(vbuf, [pos], v)
                plsc.store_scatter(bbuf, [pos], b)
                plsc.addupdate_scatter(fill_v, [w], rank, mask=last)
            return c

        lax.fori_loop(0, B // LANES // 4, scat, 0)

        def slab_body(dhi, carry2):
            d0 = dhi * 8

            for w in range(NWIN):
                buf = chunks[w % 3]
                wait_win(w)
                nxt = w + 2
                if nxt < NWIN:
                    fire(f, d0, nxt, nxt % 3)
                else:
                    wn = nxt - NWIN

                    @pl.when(dhi + 1 < dhi_hi)
                    def _(wn=wn, slot=nxt % 3):
                        fire(f, d0 + 8, wn, slot)

                start = _scalar_at(start_v, w + 1)
                n = _scalar_at(cnt_v, w + 1)
                start = jnp.minimum(start, BUFN - LANES)
                n = jnp.minimum(n, B)

                def pull(j, c, buf=buf, w=w, start=start, n=n):
                    k = jnp.minimum(start + j * LANES, BUFN - LANES)
                    v = vbuf[pl.ds(k, LANES)]
                    b = bbuf[pl.ds(k, LANES)]
                    m = (j * LANES + iota) < n
                    b = jnp.where(m, b & (B - 1), 0)
                    vrel = v - w * WIN
                    if w < NWIN - 1:
                        vrel = jnp.where(m, vrel, 0)
                        for dl in range(8):
                            val = plsc.load_gather(
                                buf, [_splat(dl), vrel], mask=m)
                            plsc.store_scatter(
                                outb, [_splat(dl), b], val, mask=m)
                    else:
                        m_in = m & (vrel < TAIL)
                        m_t = m & (vrel >= TAIL)
                        vin = jnp.where(m_in, vrel, 0)
                        vt = jnp.where(m_t, (v - VMAIN) * D + d0, 0)
                        for dl in range(8):
                            val = plsc.load_gather(
                                buf, [_splat(dl), vin], mask=m_in)
                            plsc.store_scatter(
                                outb, [_splat(dl), b], val, mask=m_in)
                            tval = plsc.load_gather(
                                tail_v, [vt + dl], mask=m_t)
                            plsc.store_scatter(
                                outb, [_splat(dl), b], tval, mask=m_t)
                    return c

                lax.fori_loop(0, (n + LANES - 1) >> 4, pull, 0)

            pltpu.sync_copy(outb, out_hbm.at[f, pl.ds(d0, 8), pl.ds(0, B)])
            return carry2

        lax.fori_loop(dhi_lo, dhi_hi, slab_body, 0)
        return carry

    lax.fori_loop(f_lo, f_hi, feature_body, 0)


def kernel(values, lengths, tables):
    del lengths
    tabs_t = tables.transpose(0, 2, 1)
    tail = tables[:, VMAIN:, :].reshape(F * VT * D)
    vals = values.reshape(F * B)
    out = _gather_kernel(vals, tabs_t, tail)
    return out.transpose(0, 2, 1)


# speedup vs baseline: 4.8045x; 1.0020x over previous
import functools

import jax
import jax.numpy as jnp
from jax import lax
from jax.experimental import pallas as pl
from jax.experimental.pallas import tpu as pltpu
from jax.experimental.pallas import tpu_sc as plsc

F = 26
B = 4096
V = 100000
D = 64

NW = 32
WIN = 3072
NWIN = 33
VMAIN = 99968
TAIL = VMAIN - (NWIN - 1) * WIN
VT = V - VMAIN
LANES = 16
BUFN = B + NWIN * (LANES - 1) + 1

_mesh = plsc.VectorSubcoreMesh(core_axis_name="c", subcore_axis_name="s")


def _splat(x):
    return jnp.full((LANES,), x, jnp.int32)


def _scalar_at(ref, i):
    return lax.reduce_max(plsc.load_gather(ref, [_splat(i)]), (0,))


@functools.partial(
    pl.kernel,
    mesh=_mesh,
    out_type=jax.ShapeDtypeStruct((F, D, B), jnp.float32),
    scratch_types=[
        pltpu.VMEM((B,), jnp.int32),
        pltpu.VMEM((BUFN,), jnp.int32),
        pltpu.VMEM((BUFN,), jnp.int32),
        pltpu.VMEM((64,), jnp.int32),
        pltpu.VMEM((64,), jnp.int32),
        pltpu.VMEM((64,), jnp.int32),
        pltpu.VMEM((8, WIN), jnp.float32),
        pltpu.VMEM((8, WIN), jnp.float32),
        pltpu.VMEM((8, WIN), jnp.float32),
        pltpu.VMEM((VT * D,), jnp.float32),
        pltpu.VMEM((8, B), jnp.float32),
        pltpu.SemaphoreType.DMA,
        pltpu.SemaphoreType.DMA,
        pltpu.SemaphoreType.DMA,
    ],
    compiler_params=pltpu.CompilerParams(
        use_tc_tiling_on_sc=True, needs_layout_passes=False
    ),
)
def _gather_kernel(values_hbm, tabs_hbm, tail_hbm, out_hbm, idxf, vbuf, bbuf,
                   cnt_v, start_v, fill_v, chunk0, chunk1, chunk2, tail_v,
                   outb, sem0, sem1, sem2):
    wid = lax.axis_index("s") * 2 + lax.axis_index("c")
    s_lo = (13 * wid) // 2
    s_hi = (13 * (wid + 1)) // 2
    f_lo = s_lo >> 3
    f_hi = (s_hi + 7) >> 3
    iota = lax.iota(jnp.int32, LANES)
    zeros = jnp.zeros((LANES,), jnp.int32)

    chunks = (chunk0, chunk1, chunk2)
    sems = (sem0, sem1, sem2)

    def fire(fv, d0v, w, slot):
        ext = WIN if w < NWIN - 1 else TAIL
        pltpu.async_copy(
            tabs_hbm.at[fv, pl.ds(d0v, 8), pl.ds(w * WIN, ext)],
            chunks[slot].at[:, pl.ds(0, ext)], sems[slot])

    def wait_win(w):
        ext = WIN if w < NWIN - 1 else TAIL
        slot = w % 3
        pltpu.make_async_copy(
            tabs_hbm.at[0, pl.ds(0, 8), pl.ds(0, ext)],
            chunks[slot].at[:, pl.ds(0, ext)], sems[slot]).wait()

    def feature_body(f, carry):
        dhi_lo = jnp.maximum(s_lo - f * 8, 0)
        dhi_hi = jnp.minimum(s_hi - f * 8, 8)

        fire(f, dhi_lo * 8, 0, 0)
        fire(f, dhi_lo * 8, 1, 1)

        pltpu.sync_copy(values_hbm.at[pl.ds(f * B, B)], idxf)
        pltpu.sync_copy(tail_hbm.at[pl.ds(f * (VT * D), VT * D)], tail_v)

        for q in range(4):
            cnt_v[pl.ds(q * 16, 16)] = zeros

        def hist(i, c):
            for u in range(4):
                v = idxf[pl.ds((i * 4 + u) * LANES, LANES)]
                w = (((v >> 10) * 21846) >> 16) + 1
                rank, last = plsc.scan_count(w)
                plsc.addupdate_scatter(cnt_v, [w], rank, mask=last)
            return c

        lax.fori_loop(0, B // LANES // 4, hist, 0)

        tot = 0
        for q in range(4):
            cq = cnt_v[pl.ds(q * 16, 16)]
            rq = (cq + (LANES - 1)) & ~(LANES - 1)
            sq = plsc.cumsum(rq) - rq + tot
            start_v[pl.ds(q * 16, 16)] = sq | (cq << 16)
            fill_v[pl.ds(q * 16, 16)] = cq * 0 + sq
            tot = tot + lax.reduce_sum(rq, (0,))

        def scat(i, c):
            for u in range(4):
                j = i * 4 + u
                v = idxf[pl.ds(j * LANES, LANES)]
                b = j * LANES + iota
                w = (((v >> 10) * 21846) >> 16) + 1
                rank, last = plsc.scan_count(w)
                base = plsc.load_gather(fill_v, [w])
                pos = base + rank - 1
                plsc.store_scatter(vbuf, [pos], v)
                plsc.store_scatter(bbuf, [pos], b)
                plsc.addupdate_scatter(fill_v, [w], rank, mask=last)
            return c

        lax.fori_loop(0, B // LANES // 4, scat, 0)

        def slab_body(dhi, carry2):
            d0 = dhi * 8

            for w in range(NWIN):
                buf = chunks[w % 3]
                wait_win(w)
                nxt = w + 2
                if nxt < NWIN:
                    fire(f, d0, nxt, nxt % 3)
                else:
                    wn = nxt - NWIN

                    @pl.when(dhi + 1 < dhi_hi)
                    def _(wn=wn, slot=nxt % 3):
                        fire(f, d0 + 8, wn, slot)

                p = _scalar_at(start_v, w + 1)
                start = jnp.minimum(p & 0xFFFF, BUFN - LANES)
                n = jnp.minimum(p >> 16, B)

                def pull(j, c, buf=buf, w=w, start=start, n=n):
                    k = jnp.minimum(start + j * LANES, BUFN - LANES)
                    v = vbuf[pl.ds(k, LANES)]
                    b = bbuf[pl.ds(k, LANES)]
                    m = (j * LANES + iota) < n
                    b = jnp.where(m, b & (B - 1), 0)
                    vrel = v - w * WIN
                    if w < NWIN - 1:
                        vrel = jnp.where(m, vrel, 0)
                        for dl in range(8):
                            val = plsc.load_gather(
                                buf, [_splat(dl), vrel], mask=m)
                            plsc.store_scatter(
                                outb, [_splat(dl), b], val, mask=m)
                    else:
                        m_in = m & (vrel < TAIL)
                        m_t = m & (vrel >= TAIL)
                        vin = jnp.where(m_in, vrel, 0)
                        vt = jnp.where(m_t, (v - VMAIN) * D + d0, 0)
                        for dl in range(8):
                            val = plsc.load_gather(
                                buf, [_splat(dl), vin], mask=m_in)
                            plsc.store_scatter(
                                outb, [_splat(dl), b], val, mask=m_in)
                            tval = plsc.load_gather(
                                tail_v, [vt + dl], mask=m_t)
                            plsc.store_scatter(
                                outb, [_splat(dl), b], tval, mask=m_t)
                    return c

                lax.fori_loop(0, (n + LANES - 1) >> 4, pull, 0)

            pltpu.sync_copy(outb, out_hbm.at[f, pl.ds(d0, 8), pl.ds(0, B)])
            return carry2

        lax.fori_loop(dhi_lo, dhi_hi, slab_body, 0)
        return carry

    lax.fori_loop(f_lo, f_hi, feature_body, 0)


def kernel(values, lengths, tables):
    del lengths
    tabs_t = tables.transpose(0, 2, 1)
    tail = tables[:, VMAIN:, :].reshape(F * VT * D)
    vals = values.reshape(F * B)
    out = _gather_kernel(vals, tabs_t, tail)
    return out.transpose(0, 2, 1)
